# Initial kernel scaffold; baseline (speedup 1.0000x reference)
import jax
import jax.numpy as jnp
from jax.experimental import pallas as pl


def kernel(user_embed, item_embed, adj_indices, adj_values):
    raise NotImplementedError("write your pallas kernel here")


# speedup vs baseline: 4.0795x; 4.0795x over previous
import functools

import jax
import jax.numpy as jnp
from jax import lax
from jax.experimental import pallas as pl
from jax.experimental.pallas import tpu as pltpu
from jax.experimental.pallas import tpu_sc as plsc

_N_USERS = 4000
_N_ITEMS = 6000
_N = _N_USERS + _N_ITEMS
_E = 320000
_D = 128
_HOPS = 3

_NC = 2
_NS = 16
_NW = _NC * _NS
_C = 128
_NCH = -(-_E // (_NW * _C))
_EPAD = _NW * _NCH * _C
_RPS = 632
_NPAD = _NS * _RPS

_mesh = plsc.VectorSubcoreMesh(core_axis_name="core", subcore_axis_name="subcore")


@functools.partial(
    pl.kernel,
    mesh=_mesh,
    out_type=jax.ShapeDtypeStruct((_NC, _NPAD, _D), jnp.float32),
    scratch_types=[
        pltpu.VMEM_SHARED((_NPAD, _D), jnp.float32),
        pltpu.VMEM((_NCH, _C), jnp.int32),
        pltpu.VMEM((_NCH, _C), jnp.int32),
        pltpu.VMEM((_NCH, _C), jnp.float32),
        pltpu.VMEM((_C, _D), jnp.float32),
    ],
)
def _hop(src_hbm, col_hbm, row_hbm, val_hbm, out_hbm,
         acc, col_v, row_v, val_v, rows_v):
    c = lax.axis_index("core")
    s = lax.axis_index("subcore")
    wid = c * _NS + s

    @pl.loop(0, _C)
    def _(i):
        @pl.loop(0, _D, step=16)
        def _(j):
            rows_v[i, pl.ds(j, 16)] = jnp.zeros((16,), jnp.float32)

    for k in range(4):
        pltpu.sync_copy(rows_v, acc.at[pl.ds(s * _RPS + k * _C, _C)])
    pltpu.sync_copy(rows_v.at[pl.ds(0, _RPS - 4 * _C)],
                    acc.at[pl.ds(s * _RPS + 4 * _C, _RPS - 4 * _C)])

    pltpu.sync_copy(col_hbm.at[wid], col_v)
    pltpu.sync_copy(row_hbm.at[wid], row_v)
    pltpu.sync_copy(val_hbm.at[wid], val_v)

    plsc.subcore_barrier()

    @pl.loop(0, _NCH)
    def _(j):
        pltpu.sync_copy(src_hbm.at[col_v.at[j]], rows_v)

        @pl.loop(0, _C, step=16)
        def _(e0):
            vv = val_v[j, pl.ds(e0, 16)]
            for l in range(16):
                v = vv[l]
                for sub in range(_D // 16):
                    sl = pl.ds(sub * 16, 16)
                    rows_v[e0 + l, sl] = rows_v[e0 + l, sl] * v

        pltpu.sync_copy(rows_v, acc.at[row_v.at[j]], add=True)

    plsc.subcore_barrier()

    pltpu.sync_copy(acc.at[pl.ds(s * _RPS, _RPS)],
                    out_hbm.at[c, pl.ds(s * _RPS, _RPS)])


def _add_body(p_ref, o_ref):
    o_ref[...] = p_ref[0] + p_ref[1]


_BLK = 632


def _combine(parts):
    return pl.pallas_call(
        _add_body,
        grid=(_NPAD // _BLK,),
        in_specs=[pl.BlockSpec((_NC, _BLK, _D), lambda i: (0, i, 0))],
        out_specs=pl.BlockSpec((_BLK, _D), lambda i: (i, 0)),
        out_shape=jax.ShapeDtypeStruct((_NPAD, _D), jnp.float32),
    )(parts)


def kernel(user_embed, item_embed, adj_indices, adj_values):
    x = jnp.concatenate([user_embed, item_embed], axis=0)
    pad = _EPAD - _E
    row = jnp.concatenate([adj_indices[0], jnp.zeros((pad,), jnp.int32)])
    col = jnp.concatenate([adj_indices[1], jnp.zeros((pad,), jnp.int32)])
    val = jnp.concatenate([adj_values, jnp.zeros((pad,), jnp.float32)])
    row = row.reshape(_NW, _NCH, _C)
    col = col.reshape(_NW, _NCH, _C)
    val = val.reshape(_NW, _NCH, _C)

    embs = [x]
    for _ in range(_HOPS):
        parts = _hop(x, col, row, val)
        x = _combine(parts)[: _N]
        embs.append(x)
    embs = jnp.stack(embs, axis=1)
    return embs[:_N_USERS], embs[_N_USERS:]
